# baseline (speedup 1.0000x reference)
import jax
import jax.numpy as jnp
from jax.experimental import pallas as pl
from jax.experimental.pallas import tpu as pltpu

B = 32
P = 8732
C = 21
NOBJ = 8
G = 16
STEPS = B // G
VAR0, VAR1 = 0.1, 0.2
THRESHOLD = 0.5
NEGPOS_RATIO = 3


def _body(loc_ref, conf_ref, pri_ref, tgt_ref, out_ref, u_s, pos_s, ll_s):
    g = pl.program_id(0)

    @pl.when(g == 0)
    def _init():
        ll_s[0] = 0.0

    tt = tgt_ref[...].reshape(NOBJ * G, 5)
    tx1 = tt[:, 0:1]
    ty1 = tt[:, 1:2]
    tx2 = tt[:, 2:3]
    ty2 = tt[:, 3:4]
    lblc = tt[:, 4:5]

    pr = pri_ref[...]
    pcx = pr[0:1]
    pcy = pr[1:2]
    pw = pr[2:3]
    ph = pr[3:4]
    px1 = pcx - pw * 0.5
    py1 = pcy - ph * 0.5
    px2 = pcx + pw * 0.5
    py2 = pcy + ph * 0.5

    iw = jnp.clip(jnp.minimum(tx2, px2) - jnp.maximum(tx1, px1), 0.0, None)
    ih = jnp.clip(jnp.minimum(ty2, py2) - jnp.maximum(ty1, py1), 0.0, None)
    inter = iw * ih
    area_t = (tx2 - tx1) * (ty2 - ty1)
    area_p = (px2 - px1) * (py2 - py1)
    ov = inter / (area_t + area_p - inter)

    bto = ov[0:G]
    bti = jnp.zeros((G, P), dtype=jnp.int32)
    for j in range(1, NOBJ):
        ovj = ov[j * G:(j + 1) * G]
        m = ovj > bto
        bto = jnp.where(m, ovj, bto)
        bti = jnp.where(m, j, bti)

    rowmax = jnp.max(ov, axis=1, keepdims=True)
    iop_2d = jax.lax.broadcasted_iota(jnp.int32, (NOBJ * G, P), 1)
    cand = jnp.where(ov == rowmax, iop_2d, P)
    bpi = jnp.min(cand, axis=1, keepdims=True)

    iop = jax.lax.broadcasted_iota(jnp.int32, (1, P), 1)
    mj = jnp.full((G, P), -1, dtype=jnp.int32)
    for j in range(NOBJ):
        mj = jnp.where(iop == bpi[j * G:(j + 1) * G], j, mj)
    hasm = mj >= 0
    bti = jnp.where(hasm, mj, bti)
    bto = jnp.where(hasm, 2.0, bto)

    mx1 = tx1[0:G] + jnp.zeros((G, P), jnp.float32)
    my1 = ty1[0:G] + jnp.zeros((G, P), jnp.float32)
    mx2 = tx2[0:G] + jnp.zeros((G, P), jnp.float32)
    my2 = ty2[0:G] + jnp.zeros((G, P), jnp.float32)
    mlb = lblc[0:G] + jnp.zeros((G, P), jnp.float32)
    for j in range(1, NOBJ):
        sel = bti == j
        sl = slice(j * G, (j + 1) * G)
        mx1 = jnp.where(sel, tx1[sl], mx1)
        my1 = jnp.where(sel, ty1[sl], my1)
        mx2 = jnp.where(sel, tx2[sl], mx2)
        my2 = jnp.where(sel, ty2[sl], my2)
        mlb = jnp.where(sel, lblc[sl], mlb)

    conf_t = mlb.astype(jnp.int32) + 1
    conf_t = jnp.where(bto < THRESHOLD, 0, conf_t)
    posf = (conf_t > 0).astype(jnp.float32)

    g_cx = ((mx1 + mx2) * 0.5 - pcx) / (VAR0 * pw)
    g_cy = ((my1 + my2) * 0.5 - pcy) / (VAR0 * ph)
    g_w = jnp.log((mx2 - mx1) / pw) / VAR1
    g_h = jnp.log((my2 - my1) / ph) / VAR1

    ll_row = jnp.float32(0.0)
    for row, gv in ((0, g_cx), (1, g_cy), (2, g_w), (3, g_h)):
        d = loc_ref[row] - gv
        ad = jnp.abs(d)
        sl1 = jnp.where(ad < 1.0, 0.5 * d * d, ad - 0.5)
        ll_row = ll_row + jnp.sum(sl1 * posf)
    ll_s[0] += ll_row

    ssum = jnp.zeros((G, P), jnp.float32)
    gath = jnp.zeros((G, P), jnp.float32)
    for c in range(C):
        cfc = conf_ref[c]
        ssum = ssum + jnp.exp(cfc)
        gath = jnp.where(conf_t == c, cfc, gath)
    u = jnp.log(ssum) - gath

    u_s[pl.ds(g * G, G), :] = u
    pos_s[pl.ds(g * G, G), :] = posf

    @pl.when(g == STEPS - 1)
    def _finish():
        uu = u_s[...]
        pf = pos_s[...]
        npos = jnp.sum(pf, axis=1, keepdims=True)
        kf = jnp.minimum(jnp.float32(NEGPOS_RATIO) * npos,
                         jnp.float32(P - 1))
        um = jnp.maximum(uu * (1.0 - pf), 0.0)
        ub = jax.lax.bitcast_convert_type(um, jnp.int32)

        lo_b = jnp.zeros((B, 1), dtype=jnp.int32)
        hi_b = jnp.full((B, 1), 0x7F800000, dtype=jnp.int32)
        for _ in range(31):
            mid = lo_b + (hi_b - lo_b) // 2
            cnt = jnp.sum((ub >= mid).astype(jnp.float32), axis=1,
                          keepdims=True)
            cond = cnt >= kf
            lo_b = jnp.where(cond, mid, lo_b)
            hi_b = jnp.where(cond, hi_b, mid)
        sel = (ub >= lo_b).astype(jnp.float32)

        lc = jnp.sum(uu * pf) + jnp.sum(um * sel)
        n_tot = jnp.sum(npos)
        ll_n = ll_s[0] / n_tot
        lc_n = lc / n_tot
        lane = jax.lax.broadcasted_iota(jnp.int32, (1, 128), 1)
        out_ref[...] = jnp.where(
            lane == 0, ll_n,
            jnp.where(lane == 1, lc_n, ll_n + lc_n))


def _run(loc_r, conf_r, priors_r, tgt_r):
    out = pl.pallas_call(
        _body,
        grid=(STEPS,),
        in_specs=[
            pl.BlockSpec((4, G, P), lambda g: (0, g, 0)),
            pl.BlockSpec((C, G, P), lambda g: (0, g, 0)),
            pl.BlockSpec((4, P), lambda g: (0, 0)),
            pl.BlockSpec((NOBJ, G, 5), lambda g: (0, g, 0)),
        ],
        out_specs=pl.BlockSpec((1, 128), lambda g: (0, 0)),
        out_shape=jax.ShapeDtypeStruct((1, 128), jnp.float32),
        scratch_shapes=[
            pltpu.VMEM((B, P), jnp.float32),
            pltpu.VMEM((B, P), jnp.float32),
            pltpu.SMEM((1,), jnp.float32),
        ],
    )(loc_r, conf_r, priors_r, tgt_r)
    return out[0, 0], out[0, 1], out[0, 2]


def kernel(loc_data, conf_data, priors, targets):
    loc_r = jnp.transpose(loc_data, (2, 0, 1))
    conf_r = jnp.transpose(conf_data, (2, 0, 1))
    priors_r = jnp.transpose(priors[:P, :], (1, 0))
    tgt_r = jnp.transpose(targets, (1, 0, 2))
    return _run(loc_r, conf_r, priors_r, tgt_r)


# speedup vs baseline: 1.0512x; 1.0512x over previous
import jax
import jax.numpy as jnp
from jax.experimental import pallas as pl
from jax.experimental.pallas import tpu as pltpu

B = 32
P = 8732
C = 21
NOBJ = 8
G = 8
STEPS = B // G
VAR0, VAR1 = 0.1, 0.2
THRESHOLD = 0.5
NEGPOS_RATIO = 3


def _body(loc_ref, conf_ref, pri_ref, tgt_ref, out_ref, u_s, pos_s, ll_s):
    g = pl.program_id(0)

    @pl.when(g == 0)
    def _init():
        ll_s[0] = 0.0

    tt = tgt_ref[...].reshape(NOBJ * G, 5)
    tx1 = tt[:, 0:1]
    ty1 = tt[:, 1:2]
    tx2 = tt[:, 2:3]
    ty2 = tt[:, 3:4]
    lblc = tt[:, 4:5]

    pr = pri_ref[...]
    pcx = pr[0:1]
    pcy = pr[1:2]
    pw = pr[2:3]
    ph = pr[3:4]
    px1 = pcx - pw * 0.5
    py1 = pcy - ph * 0.5
    px2 = pcx + pw * 0.5
    py2 = pcy + ph * 0.5

    iw = jnp.clip(jnp.minimum(tx2, px2) - jnp.maximum(tx1, px1), 0.0, None)
    ih = jnp.clip(jnp.minimum(ty2, py2) - jnp.maximum(ty1, py1), 0.0, None)
    inter = iw * ih
    area_t = (tx2 - tx1) * (ty2 - ty1)
    area_p = (px2 - px1) * (py2 - py1)
    ov = inter / (area_t + area_p - inter)

    bto = ov[0:G]
    bti = jnp.zeros((G, P), dtype=jnp.int32)
    for j in range(1, NOBJ):
        ovj = ov[j * G:(j + 1) * G]
        m = ovj > bto
        bto = jnp.where(m, ovj, bto)
        bti = jnp.where(m, j, bti)

    rowmax = jnp.max(ov, axis=1, keepdims=True)
    iop_2d = jax.lax.broadcasted_iota(jnp.int32, (NOBJ * G, P), 1)
    cand = jnp.where(ov == rowmax, iop_2d, P)
    bpi = jnp.min(cand, axis=1, keepdims=True)

    iop = jax.lax.broadcasted_iota(jnp.int32, (1, P), 1)
    mj = jnp.full((G, P), -1, dtype=jnp.int32)
    for j in range(NOBJ):
        mj = jnp.where(iop == bpi[j * G:(j + 1) * G], j, mj)
    hasm = mj >= 0
    bti = jnp.where(hasm, mj, bti)
    bto = jnp.where(hasm, 2.0, bto)

    mx1 = tx1[0:G] + jnp.zeros((G, P), jnp.float32)
    my1 = ty1[0:G] + jnp.zeros((G, P), jnp.float32)
    mx2 = tx2[0:G] + jnp.zeros((G, P), jnp.float32)
    my2 = ty2[0:G] + jnp.zeros((G, P), jnp.float32)
    mlb = lblc[0:G] + jnp.zeros((G, P), jnp.float32)
    for j in range(1, NOBJ):
        sel = bti == j
        sl = slice(j * G, (j + 1) * G)
        mx1 = jnp.where(sel, tx1[sl], mx1)
        my1 = jnp.where(sel, ty1[sl], my1)
        mx2 = jnp.where(sel, tx2[sl], mx2)
        my2 = jnp.where(sel, ty2[sl], my2)
        mlb = jnp.where(sel, lblc[sl], mlb)

    conf_t = mlb.astype(jnp.int32) + 1
    conf_t = jnp.where(bto < THRESHOLD, 0, conf_t)
    posf = (conf_t > 0).astype(jnp.float32)

    g_cx = ((mx1 + mx2) * 0.5 - pcx) / (VAR0 * pw)
    g_cy = ((my1 + my2) * 0.5 - pcy) / (VAR0 * ph)
    g_w = jnp.log((mx2 - mx1) / pw) / VAR1
    g_h = jnp.log((my2 - my1) / ph) / VAR1

    ll_row = jnp.float32(0.0)
    for row, gv in ((0, g_cx), (1, g_cy), (2, g_w), (3, g_h)):
        d = loc_ref[row] - gv
        ad = jnp.abs(d)
        sl1 = jnp.where(ad < 1.0, 0.5 * d * d, ad - 0.5)
        ll_row = ll_row + jnp.sum(sl1 * posf)
    ll_s[0] += ll_row

    ssum = jnp.zeros((G, P), jnp.float32)
    gath = jnp.zeros((G, P), jnp.float32)
    for c in range(C):
        cfc = conf_ref[c]
        ssum = ssum + jnp.exp(cfc)
        gath = jnp.where(conf_t == c, cfc, gath)
    u = jnp.log(ssum) - gath

    u_s[pl.ds(g * G, G), :] = u
    pos_s[pl.ds(g * G, G), :] = posf

    @pl.when(g == STEPS - 1)
    def _finish():
        uu = u_s[...]
        pf = pos_s[...]
        npos = jnp.sum(pf, axis=1, keepdims=True)
        kf = jnp.minimum(jnp.float32(NEGPOS_RATIO) * npos,
                         jnp.float32(P - 1))
        um = jnp.maximum(uu * (1.0 - pf), 0.0)
        ub = jax.lax.bitcast_convert_type(um, jnp.int32)

        lo_b = jnp.zeros((B, 1), dtype=jnp.int32)
        hi_b = jnp.full((B, 1), 0x7F800000, dtype=jnp.int32)
        for _ in range(31):
            mid = lo_b + (hi_b - lo_b) // 2
            cnt = jnp.sum((ub >= mid).astype(jnp.float32), axis=1,
                          keepdims=True)
            cond = cnt >= kf
            lo_b = jnp.where(cond, mid, lo_b)
            hi_b = jnp.where(cond, hi_b, mid)
        sel = (ub >= lo_b).astype(jnp.float32)

        lc = jnp.sum(uu * pf) + jnp.sum(um * sel)
        n_tot = jnp.sum(npos)
        ll_n = ll_s[0] / n_tot
        lc_n = lc / n_tot
        lane = jax.lax.broadcasted_iota(jnp.int32, (1, 128), 1)
        out_ref[...] = jnp.where(
            lane == 0, ll_n,
            jnp.where(lane == 1, lc_n, ll_n + lc_n))


def _run(loc_r, conf_r, priors_r, tgt_r):
    out = pl.pallas_call(
        _body,
        grid=(STEPS,),
        in_specs=[
            pl.BlockSpec((4, G, P), lambda g: (0, g, 0)),
            pl.BlockSpec((C, G, P), lambda g: (0, g, 0)),
            pl.BlockSpec((4, P), lambda g: (0, 0)),
            pl.BlockSpec((NOBJ, G, 5), lambda g: (0, g, 0)),
        ],
        out_specs=pl.BlockSpec((1, 128), lambda g: (0, 0)),
        out_shape=jax.ShapeDtypeStruct((1, 128), jnp.float32),
        scratch_shapes=[
            pltpu.VMEM((B, P), jnp.float32),
            pltpu.VMEM((B, P), jnp.float32),
            pltpu.SMEM((1,), jnp.float32),
        ],
    )(loc_r, conf_r, priors_r, tgt_r)
    return out[0, 0], out[0, 1], out[0, 2]


def kernel(loc_data, conf_data, priors, targets):
    loc_r = jnp.transpose(loc_data, (2, 0, 1))
    conf_r = jnp.transpose(conf_data, (2, 0, 1))
    priors_r = jnp.transpose(priors[:P, :], (1, 0))
    tgt_r = jnp.transpose(targets, (1, 0, 2))
    return _run(loc_r, conf_r, priors_r, tgt_r)
